# baseline (speedup 1.0000x reference)
import jax
import jax.numpy as jnp
from jax import lax
from jax.experimental import pallas as pl
from jax.experimental.pallas import tpu as pltpu
from jax.experimental.pallas import tpu_sc as plsc

N = 512
D = 512
NQ = 4
QW = D // NQ
NB = QW // 32
NG = N // 4
RPT = 64
SC = 4
GPS = NG // SC

_info = plsc.get_sparse_core_info()
_NC, _NS = _info.num_cores, _info.num_subcores
_NW = _NC * _NS


def _prep_body(x_ref, a_ref, w_ref, b_ref, h_ref, tab_ref, deg_ref, m_ref):
    x = x_ref[...]
    a = a_ref[...].astype(jnp.float32)
    w = w_ref[...]
    b = b_ref[...]
    h_ref[...] = jax.lax.dot_general(
        x, w, (((1,), (1,)), ((), ())), preferred_element_type=jnp.float32
    ) + b
    tab_ref[...] = jax.lax.dot_general(
        w, x, (((1,), (1,)), ((), ())), preferred_element_type=jnp.float32
    ) + b[0][:, None]
    deg_ref[...] = jnp.sum(a, axis=1, keepdims=True).T
    ci = lax.broadcasted_iota(jnp.int32, (N, NG), 0)
    gi = lax.broadcasted_iota(jnp.int32, (N, NG), 1)
    sel = jnp.where(ci // 4 == gi, 1 << (ci % 4), 0).astype(jnp.float32)
    m = jax.lax.dot_general(
        a, sel, (((1,), (0,)), ((), ())), preferred_element_type=jnp.float32
    )
    m_ref[...] = m.astype(jnp.int32)


def _sc_body(tab_hbm, m_hbm, h_hbm, deg_hbm, out_hbm,
             trows_v, tbl_v, m_v, h_v, deg_v, out_v, sems):
    cid = lax.axis_index("c")
    sid = lax.axis_index("s")
    wid = sid * _NC + cid
    q = wid % NQ
    qoff = pl.multiple_of(q * QW, QW)
    rows0 = (wid // NQ) * RPT
    pltpu.sync_copy(m_hbm.at[pl.ds(rows0, RPT)], m_v)
    pltpu.sync_copy(h_hbm.at[pl.ds(rows0, RPT), pl.ds(qoff, QW)], h_v)
    pltpu.sync_copy(deg_hbm, deg_v)

    _half = jnp.int32(0x8000)
    _hmask = jnp.int32(-65536)

    def _pack2(lo, hi):
        lo_b = lax.shift_right_logical(
            lax.bitcast_convert_type(lo, jnp.int32), 16)
        hi_b = lax.bitcast_convert_type(hi, jnp.int32) & _hmask
        return lo_b | hi_b

    def _mk_copy(sc, buf):
        return pltpu.async_copy(
            tab_hbm.at[pl.ds(sc * (GPS * 4), GPS * 4), pl.ds(qoff, QW)],
            trows_v.at[pl.ds(buf * (GPS * 4), GPS * 4)],
            sems.at[buf])

    def ztbl(g, _):
        for f in range(NB):
            tbl_v[16 * g, pl.ds(16 * f, 16)] = jnp.zeros((16,), jnp.int32)
        return 0

    lax.fori_loop(0, GPS, ztbl, 0)

    copies = [None] * SC
    copies[0] = _mk_copy(0, 0)
    for sc in range(SC):
        if sc + 1 < SC:
            copies[sc + 1] = _mk_copy(sc + 1, (sc + 1) % 2)
        copies[sc].wait()
        boff = (sc % 2) * (GPS * 4)

        def build(g, _, boff=boff):
            for f in range(NB):
                sl0 = pl.ds(32 * f, 16)
                sl1 = pl.ds(32 * f + 16, 16)
                psl = pl.ds(16 * f, 16)
                rows = [(trows_v[boff + g * 4 + k, sl0],
                         trows_v[boff + g * 4 + k, sl1])
                        for k in range(4)]
                ent = [None] * 16
                ent[1] = rows[0]
                ent[2] = rows[1]
                ent[3] = (rows[0][0] + rows[1][0], rows[0][1] + rows[1][1])
                ent[4] = rows[2]
                ent[8] = rows[3]
                ent[12] = (rows[2][0] + rows[3][0], rows[2][1] + rows[3][1])
                for hibits in (4, 8, 12):
                    hi = ent[hibits]
                    for lobits in (1, 2, 3):
                        lo = ent[lobits]
                        ent[hibits + lobits] = (lo[0] + hi[0], lo[1] + hi[1])
                for msk in range(1, 16):
                    tbl_v[16 * g + msk, psl] = _pack2(*ent[msk])
            return 0

        lax.fori_loop(0, GPS, build, 0)

        def lk(r, _, sc=sc):
            mvs = [m_v[r, pl.ds(sc * GPS + 16 * i, 16)]
                   for i in range(GPS // 16)]
            if sc == 0:
                acc = (jnp.zeros((16,), jnp.float32),) * (QW // 16)
            else:
                acc = tuple(out_v[r, pl.ds(16 * f, 16)]
                            for f in range(QW // 16))
            for g in range(GPS):
                idx = 16 * g + mvs[g // 16][g % 16]
                for f in range(NB):
                    e = tbl_v[idx, pl.ds(16 * f, 16)]
                    lo = lax.bitcast_convert_type(
                        lax.shift_left(e, 16), jnp.float32)
                    hi = lax.bitcast_convert_type(e, jnp.float32)
                    acc = (acc[:2 * f]
                           + (acc[2 * f] + lo, acc[2 * f + 1] + hi)
                           + acc[2 * f + 2:])
            for f in range(QW // 16):
                out_v[r, pl.ds(16 * f, 16)] = acc[f]
            return 0

        lax.fori_loop(0, RPT, lk, 0, unroll=2)

    def comb(r, _):
        for f in range(QW // 16):
            sl = pl.ds(16 * f, 16)
            out_v[r, sl] = (out_v[r, sl] + h_v[r, sl]) / \
                deg_v[pl.ds(qoff + 16 * f, 16)]
        return 0

    lax.fori_loop(0, RPT, comb, 0)
    pltpu.sync_copy(out_v, out_hbm.at[pl.ds(rows0, RPT), pl.ds(qoff, QW)])


_agg = pl.kernel(
    _sc_body,
    out_type=jax.ShapeDtypeStruct((N, D), jnp.float32),
    mesh=plsc.VectorSubcoreMesh(core_axis_name="c", subcore_axis_name="s"),
    scratch_types=[
        pltpu.VMEM((2 * GPS * 4, QW), jnp.float32),
        pltpu.VMEM((GPS * 16, QW // 2), jnp.int32),
        pltpu.VMEM((RPT, NG), jnp.int32),
        pltpu.VMEM((RPT, QW), jnp.float32),
        pltpu.VMEM((N,), jnp.float32),
        pltpu.VMEM((RPT, QW), jnp.float32),
        pltpu.SemaphoreType.DMA((2,)),
    ],
)


def kernel(node_features, adj_matrix, W, b):
    x = node_features[0]
    a = adj_matrix[0].astype(jnp.int32)
    h, tab, deg, m = pl.pallas_call(
        _prep_body,
        out_shape=[
            jax.ShapeDtypeStruct((N, D), jnp.float32),
            jax.ShapeDtypeStruct((N, D), jnp.float32),
            jax.ShapeDtypeStruct((1, N), jnp.float32),
            jax.ShapeDtypeStruct((N, NG), jnp.int32),
        ],
    )(x, a, W, b.reshape(1, -1))
    out = _agg(tab, m, h, deg.reshape(-1))
    return out[None]


# speedup vs baseline: 1.0193x; 1.0193x over previous
import jax
import jax.numpy as jnp
from jax import lax
from jax.experimental import pallas as pl
from jax.experimental.pallas import tpu as pltpu
from jax.experimental.pallas import tpu_sc as plsc

N = 512
D = 512
NQ = 4
QW = D // NQ
NB = QW // 32
NG = N // 4
RPT = 64
SC = 4
GPS = NG // SC

_info = plsc.get_sparse_core_info()
_NC, _NS = _info.num_cores, _info.num_subcores
_NW = _NC * _NS


def _prep_body(x_ref, a_ref, w_ref, b_ref, h_ref, tab_ref, deg_ref, m_ref):
    x = x_ref[...]
    a = a_ref[...].astype(jnp.float32)
    w = w_ref[...]
    b = b_ref[...]
    h_ref[...] = jax.lax.dot_general(
        x, w, (((1,), (1,)), ((), ())), preferred_element_type=jnp.float32
    ) + b
    tab_ref[...] = jax.lax.dot_general(
        w, x, (((1,), (1,)), ((), ())), preferred_element_type=jnp.float32
    ) + b[0][:, None]
    deg_ref[...] = jnp.sum(a, axis=1, keepdims=True).T
    ci = lax.broadcasted_iota(jnp.int32, (N, NG), 0)
    gi = lax.broadcasted_iota(jnp.int32, (N, NG), 1)
    sel = jnp.where(ci // 4 == gi, 1 << (ci % 4), 0).astype(jnp.float32)
    m = jax.lax.dot_general(
        a, sel, (((1,), (0,)), ((), ())), preferred_element_type=jnp.float32
    )
    m_ref[...] = m.astype(jnp.int32)


def _sc_body(tab_hbm, m_hbm, h_hbm, deg_hbm, out_hbm,
             trows_v, tbl_v, m_v, h_v, deg_v, out_v, sems):
    cid = lax.axis_index("c")
    sid = lax.axis_index("s")
    wid = sid * _NC + cid
    q = wid % NQ
    qoff = pl.multiple_of(q * QW, QW)
    rows0 = (wid // NQ) * RPT
    pltpu.sync_copy(m_hbm.at[pl.ds(rows0, RPT)], m_v)
    pltpu.sync_copy(h_hbm.at[pl.ds(rows0, RPT), pl.ds(qoff, QW)], h_v)
    pltpu.sync_copy(deg_hbm, deg_v)

    _half = jnp.int32(0x8000)
    _hmask = jnp.int32(-65536)

    def _pack2(lo, hi):
        lo_b = lax.shift_right_logical(
            lax.bitcast_convert_type(lo, jnp.int32), 16)
        hi_b = lax.bitcast_convert_type(hi, jnp.int32) & _hmask
        return lo_b | hi_b

    def _mk_copy(sc, buf):
        return pltpu.async_copy(
            tab_hbm.at[pl.ds(sc * (GPS * 4), GPS * 4), pl.ds(qoff, QW)],
            trows_v.at[pl.ds(buf * (GPS * 4), GPS * 4)],
            sems.at[buf])

    def ztbl(g, _):
        for f in range(NB):
            tbl_v[16 * g, pl.ds(16 * f, 16)] = jnp.zeros((16,), jnp.int32)
        return 0

    lax.fori_loop(0, GPS, ztbl, 0)

    copies = [None] * SC
    copies[0] = _mk_copy(0, 0)
    for sc in range(SC):
        if sc + 1 < SC:
            copies[sc + 1] = _mk_copy(sc + 1, (sc + 1) % 2)
        copies[sc].wait()
        boff = (sc % 2) * (GPS * 4)

        def build(g, _, boff=boff):
            for f in range(NB):
                sl0 = pl.ds(32 * f, 16)
                sl1 = pl.ds(32 * f + 16, 16)
                psl = pl.ds(16 * f, 16)
                rows = [(trows_v[boff + g * 4 + k, sl0],
                         trows_v[boff + g * 4 + k, sl1])
                        for k in range(4)]
                ent = [None] * 16
                ent[1] = rows[0]
                ent[2] = rows[1]
                ent[3] = (rows[0][0] + rows[1][0], rows[0][1] + rows[1][1])
                ent[4] = rows[2]
                ent[8] = rows[3]
                ent[12] = (rows[2][0] + rows[3][0], rows[2][1] + rows[3][1])
                for hibits in (4, 8, 12):
                    hi = ent[hibits]
                    for lobits in (1, 2, 3):
                        lo = ent[lobits]
                        ent[hibits + lobits] = (lo[0] + hi[0], lo[1] + hi[1])
                for msk in range(1, 16):
                    tbl_v[16 * g + msk, psl] = _pack2(*ent[msk])
            return 0

        lax.fori_loop(0, GPS, build, 0)

        def lk(r, _, sc=sc):
            mvs = [m_v[r, pl.ds(sc * GPS + 16 * i, 16)]
                   for i in range(GPS // 16)]
            if sc == 0:
                acc = (jnp.zeros((16,), jnp.float32),) * (QW // 16)
            else:
                acc = tuple(out_v[r, pl.ds(16 * f, 16)]
                            for f in range(QW // 16))
            for g in range(GPS):
                idx = 16 * g + mvs[g // 16][g % 16]
                for f in range(NB):
                    e = tbl_v[idx, pl.ds(16 * f, 16)]
                    lo = lax.bitcast_convert_type(
                        lax.shift_left(e, 16), jnp.float32)
                    hi = lax.bitcast_convert_type(e, jnp.float32)
                    acc = (acc[:2 * f]
                           + (acc[2 * f] + lo, acc[2 * f + 1] + hi)
                           + acc[2 * f + 2:])
            for f in range(QW // 16):
                out_v[r, pl.ds(16 * f, 16)] = acc[f]
            return 0

        lax.fori_loop(0, RPT, lk, 0)

    def comb(r, _):
        for f in range(QW // 16):
            sl = pl.ds(16 * f, 16)
            out_v[r, sl] = (out_v[r, sl] + h_v[r, sl]) / \
                deg_v[pl.ds(qoff + 16 * f, 16)]
        return 0

    lax.fori_loop(0, RPT, comb, 0)
    pltpu.sync_copy(out_v, out_hbm.at[pl.ds(rows0, RPT), pl.ds(qoff, QW)])


_agg = pl.kernel(
    _sc_body,
    out_type=jax.ShapeDtypeStruct((N, D), jnp.float32),
    mesh=plsc.VectorSubcoreMesh(core_axis_name="c", subcore_axis_name="s"),
    scratch_types=[
        pltpu.VMEM((2 * GPS * 4, QW), jnp.float32),
        pltpu.VMEM((GPS * 16, QW // 2), jnp.int32),
        pltpu.VMEM((RPT, NG), jnp.int32),
        pltpu.VMEM((RPT, QW), jnp.float32),
        pltpu.VMEM((N,), jnp.float32),
        pltpu.VMEM((RPT, QW), jnp.float32),
        pltpu.SemaphoreType.DMA((2,)),
    ],
)


def kernel(node_features, adj_matrix, W, b):
    x = node_features[0]
    a = adj_matrix[0].astype(jnp.int32)
    h, tab, deg, m = pl.pallas_call(
        _prep_body,
        out_shape=[
            jax.ShapeDtypeStruct((N, D), jnp.float32),
            jax.ShapeDtypeStruct((N, D), jnp.float32),
            jax.ShapeDtypeStruct((1, N), jnp.float32),
            jax.ShapeDtypeStruct((N, NG), jnp.int32),
        ],
    )(x, a, W, b.reshape(1, -1))
    out = _agg(tab, m, h, deg.reshape(-1))
    return out[None]


# speedup vs baseline: 1.0851x; 1.0646x over previous
import jax
import jax.numpy as jnp
from jax import lax
from jax.experimental import pallas as pl
from jax.experimental.pallas import tpu as pltpu
from jax.experimental.pallas import tpu_sc as plsc

N = 512
D = 512
NQ = 4
QW = D // NQ
NB = QW // 32
NG = N // 4
RPT = 64
SC = 4
GPS = NG // SC

_info = plsc.get_sparse_core_info()
_NC, _NS = _info.num_cores, _info.num_subcores
_NW = _NC * _NS


def _prep_body(x_ref, a_ref, w_ref, b_ref, h_ref, tab_ref, deg_ref, m_ref):
    x = x_ref[...]
    a = a_ref[...].astype(jnp.float32)
    w = w_ref[...]
    b = b_ref[...]
    h_ref[...] = jax.lax.dot_general(
        x, w, (((1,), (1,)), ((), ())), preferred_element_type=jnp.float32
    ) + b
    tab_ref[...] = jax.lax.dot_general(
        w, x, (((1,), (1,)), ((), ())), preferred_element_type=jnp.float32
    ) + b[0][:, None]
    deg_ref[...] = 1.0 / jnp.sum(a, axis=1, keepdims=True).T
    ci = lax.broadcasted_iota(jnp.int32, (N, NG), 0)
    gi = lax.broadcasted_iota(jnp.int32, (N, NG), 1)
    sel = jnp.where(ci // 4 == gi, 1 << (ci % 4), 0).astype(jnp.float32)
    m = jax.lax.dot_general(
        a, sel, (((1,), (0,)), ((), ())), preferred_element_type=jnp.float32
    )
    m_ref[...] = m.astype(jnp.int32)


def _sc_body(tab_hbm, m_hbm, h_hbm, deg_hbm, out_hbm,
             trows_v, tbl_v, m_v, h_v, deg_v, out_v, sems):
    cid = lax.axis_index("c")
    sid = lax.axis_index("s")
    wid = sid * _NC + cid
    q = wid % NQ
    qoff = pl.multiple_of(q * QW, QW)
    rows0 = (wid // NQ) * RPT
    pltpu.sync_copy(m_hbm.at[pl.ds(rows0, RPT)], m_v)
    pltpu.sync_copy(h_hbm.at[pl.ds(rows0, RPT), pl.ds(qoff, QW)], h_v)
    pltpu.sync_copy(deg_hbm, deg_v)

    _half = jnp.int32(0x8000)
    _hmask = jnp.int32(-65536)

    def _pack2(lo, hi):
        lo_b = lax.shift_right_logical(
            lax.bitcast_convert_type(lo, jnp.int32), 16)
        hi_b = lax.bitcast_convert_type(hi, jnp.int32) & _hmask
        return lo_b | hi_b

    def _mk_copy(sc, buf):
        return pltpu.async_copy(
            tab_hbm.at[pl.ds(sc * (GPS * 4), GPS * 4), pl.ds(qoff, QW)],
            trows_v.at[pl.ds(buf * (GPS * 4), GPS * 4)],
            sems.at[buf])

    def ztbl(g, _):
        for f in range(NB):
            tbl_v[16 * g, pl.ds(16 * f, 16)] = jnp.zeros((16,), jnp.int32)
        return 0

    lax.fori_loop(0, GPS, ztbl, 0)

    copies = [None] * SC
    copies[0] = _mk_copy(0, 0)
    for sc in range(SC):
        if sc + 1 < SC:
            copies[sc + 1] = _mk_copy(sc + 1, (sc + 1) % 2)
        copies[sc].wait()
        boff = (sc % 2) * (GPS * 4)

        def build(g, _, boff=boff):
            for f in range(NB):
                sl0 = pl.ds(32 * f, 16)
                sl1 = pl.ds(32 * f + 16, 16)
                psl = pl.ds(16 * f, 16)
                rows = [(trows_v[boff + g * 4 + k, sl0],
                         trows_v[boff + g * 4 + k, sl1])
                        for k in range(4)]
                ent = [None] * 16
                ent[1] = rows[0]
                ent[2] = rows[1]
                ent[3] = (rows[0][0] + rows[1][0], rows[0][1] + rows[1][1])
                ent[4] = rows[2]
                ent[8] = rows[3]
                ent[12] = (rows[2][0] + rows[3][0], rows[2][1] + rows[3][1])
                for hibits in (4, 8, 12):
                    hi = ent[hibits]
                    for lobits in (1, 2, 3):
                        lo = ent[lobits]
                        ent[hibits + lobits] = (lo[0] + hi[0], lo[1] + hi[1])
                for msk in range(1, 16):
                    tbl_v[16 * g + msk, psl] = _pack2(*ent[msk])
            return 0

        lax.fori_loop(0, GPS, build, 0)

        def lk(r, _, sc=sc):
            mvs = [m_v[r, pl.ds(sc * GPS + 16 * i, 16)]
                   for i in range(GPS // 16)]
            if sc == 0:
                acc = (jnp.zeros((16,), jnp.float32),) * (QW // 16)
            else:
                acc = tuple(out_v[r, pl.ds(16 * f, 16)]
                            for f in range(QW // 16))
            for g in range(GPS):
                idx = 16 * g + mvs[g // 16][g % 16]
                for f in range(NB):
                    e = tbl_v[idx, pl.ds(16 * f, 16)]
                    lo = lax.bitcast_convert_type(
                        lax.shift_left(e, 16), jnp.float32)
                    hi = lax.bitcast_convert_type(e, jnp.float32)
                    acc = (acc[:2 * f]
                           + (acc[2 * f] + lo, acc[2 * f + 1] + hi)
                           + acc[2 * f + 2:])
            for f in range(QW // 16):
                sl = pl.ds(16 * f, 16)
                if sc == SC - 1:
                    out_v[r, sl] = (acc[f] + h_v[r, sl]) * \
                        deg_v[pl.ds(qoff + 16 * f, 16)]
                else:
                    out_v[r, sl] = acc[f]
            return 0

        lax.fori_loop(0, RPT, lk, 0)

    pltpu.sync_copy(out_v, out_hbm.at[pl.ds(rows0, RPT), pl.ds(qoff, QW)])


_agg = pl.kernel(
    _sc_body,
    out_type=jax.ShapeDtypeStruct((N, D), jnp.float32),
    mesh=plsc.VectorSubcoreMesh(core_axis_name="c", subcore_axis_name="s"),
    scratch_types=[
        pltpu.VMEM((2 * GPS * 4, QW), jnp.float32),
        pltpu.VMEM((GPS * 16, QW // 2), jnp.int32),
        pltpu.VMEM((RPT, NG), jnp.int32),
        pltpu.VMEM((RPT, QW), jnp.float32),
        pltpu.VMEM((N,), jnp.float32),
        pltpu.VMEM((RPT, QW), jnp.float32),
        pltpu.SemaphoreType.DMA((2,)),
    ],
)


def kernel(node_features, adj_matrix, W, b):
    x = node_features[0]
    a = adj_matrix[0].astype(jnp.int32)
    h, tab, deg, m = pl.pallas_call(
        _prep_body,
        out_shape=[
            jax.ShapeDtypeStruct((N, D), jnp.float32),
            jax.ShapeDtypeStruct((N, D), jnp.float32),
            jax.ShapeDtypeStruct((1, N), jnp.float32),
            jax.ShapeDtypeStruct((N, NG), jnp.int32),
        ],
    )(x, a, W, b.reshape(1, -1))
    out = _agg(tab, m, h, deg.reshape(-1))
    return out[None]


# speedup vs baseline: 1.1122x; 1.0250x over previous
import jax
import jax.numpy as jnp
from jax import lax
from jax.experimental import pallas as pl
from jax.experimental.pallas import tpu as pltpu
from jax.experimental.pallas import tpu_sc as plsc

N = 512
D = 512
NQ = 4
QW = D // NQ
NB = QW // 32
NG = N // 4
RPT = 64
SC = 4
GPS = NG // SC

_info = plsc.get_sparse_core_info()
_NC, _NS = _info.num_cores, _info.num_subcores
_NW = _NC * _NS


def _prep_body(x_ref, a_ref, w_ref, b_ref, h_ref, tab_ref, deg_ref, m_ref):
    x = x_ref[...]
    a = a_ref[...].astype(jnp.float32)
    w = w_ref[...]
    b = b_ref[...]
    h_ref[...] = jax.lax.dot_general(
        x, w, (((1,), (1,)), ((), ())), preferred_element_type=jnp.float32
    ) + b
    tab_ref[...] = jax.lax.dot_general(
        w, x, (((1,), (1,)), ((), ())), preferred_element_type=jnp.float32
    ) + b[0][:, None]
    deg_ref[...] = 1.0 / jnp.sum(a, axis=1, keepdims=True).T
    ci = lax.broadcasted_iota(jnp.int32, (N, NG), 0)
    gi = lax.broadcasted_iota(jnp.int32, (N, NG), 1)
    sel = jnp.where(ci // 4 == gi, 1 << (ci % 4), 0).astype(jnp.float32)
    m = jax.lax.dot_general(
        a, sel, (((1,), (0,)), ((), ())), preferred_element_type=jnp.float32
    )
    m_ref[...] = m.astype(jnp.int32)


def _sc_body(tab_hbm, m_hbm, h_hbm, deg_hbm, out_hbm,
             trows_v, tbl_v, m_v, h_v, deg_v, out_v, sems):
    cid = lax.axis_index("c")
    sid = lax.axis_index("s")
    wid = sid * _NC + cid
    q = wid % NQ
    qoff = pl.multiple_of(q * QW, QW)
    rows0 = (wid // NQ) * RPT

    _half = jnp.int32(0x8000)
    _hmask = jnp.int32(-65536)

    def _pack2(lo, hi):
        lo_b = lax.shift_right_logical(
            lax.bitcast_convert_type(lo, jnp.int32), 16)
        hi_b = lax.bitcast_convert_type(hi, jnp.int32) & _hmask
        return lo_b | hi_b

    def _mk_copy(sc, buf):
        return pltpu.async_copy(
            tab_hbm.at[pl.ds(sc * (GPS * 4), GPS * 4), pl.ds(qoff, QW)],
            trows_v.at[pl.ds(buf * (GPS * 4), GPS * 4)],
            sems.at[buf])

    copies = [None] * SC
    copies[0] = _mk_copy(0, 0)
    m_cpy = pltpu.async_copy(m_hbm.at[pl.ds(rows0, RPT)], m_v, sems.at[2])
    h_cpy = pltpu.async_copy(
        h_hbm.at[pl.ds(rows0, RPT), pl.ds(qoff, QW)], h_v, sems.at[3])
    pltpu.sync_copy(deg_hbm, deg_v)

    def ztbl(g, _):
        for f in range(NB):
            tbl_v[16 * g, pl.ds(16 * f, 16)] = jnp.zeros((16,), jnp.int32)
        return 0

    lax.fori_loop(0, GPS, ztbl, 0)

    for sc in range(SC):
        if sc + 1 < SC:
            copies[sc + 1] = _mk_copy(sc + 1, (sc + 1) % 2)
        copies[sc].wait()
        if sc == 0:
            m_cpy.wait()
        if sc == SC - 1:
            h_cpy.wait()
        boff = (sc % 2) * (GPS * 4)

        def build(g, _, boff=boff):
            for f in range(NB):
                sl0 = pl.ds(32 * f, 16)
                sl1 = pl.ds(32 * f + 16, 16)
                psl = pl.ds(16 * f, 16)
                rows = [(trows_v[boff + g * 4 + k, sl0],
                         trows_v[boff + g * 4 + k, sl1])
                        for k in range(4)]
                ent = [None] * 16
                ent[1] = rows[0]
                ent[2] = rows[1]
                ent[3] = (rows[0][0] + rows[1][0], rows[0][1] + rows[1][1])
                ent[4] = rows[2]
                ent[8] = rows[3]
                ent[12] = (rows[2][0] + rows[3][0], rows[2][1] + rows[3][1])
                for hibits in (4, 8, 12):
                    hi = ent[hibits]
                    for lobits in (1, 2, 3):
                        lo = ent[lobits]
                        ent[hibits + lobits] = (lo[0] + hi[0], lo[1] + hi[1])
                for msk in range(1, 16):
                    tbl_v[16 * g + msk, psl] = _pack2(*ent[msk])
            return 0

        lax.fori_loop(0, GPS, build, 0)

        def lk(r, _, sc=sc):
            mvs = [m_v[r, pl.ds(sc * GPS + 16 * i, 16)]
                   for i in range(GPS // 16)]
            if sc == 0:
                acc = (jnp.zeros((16,), jnp.float32),) * (QW // 16)
            else:
                acc = tuple(out_v[r, pl.ds(16 * f, 16)]
                            for f in range(QW // 16))
            for g in range(GPS):
                idx = 16 * g + mvs[g // 16][g % 16]
                for f in range(NB):
                    e = tbl_v[idx, pl.ds(16 * f, 16)]
                    lo = lax.bitcast_convert_type(
                        lax.shift_left(e, 16), jnp.float32)
                    hi = lax.bitcast_convert_type(e, jnp.float32)
                    acc = (acc[:2 * f]
                           + (acc[2 * f] + lo, acc[2 * f + 1] + hi)
                           + acc[2 * f + 2:])
            for f in range(QW // 16):
                sl = pl.ds(16 * f, 16)
                if sc == SC - 1:
                    out_v[r, sl] = (acc[f] + h_v[r, sl]) * \
                        deg_v[pl.ds(qoff + 16 * f, 16)]
                else:
                    out_v[r, sl] = acc[f]
            return 0

        lax.fori_loop(0, RPT, lk, 0)

    pltpu.sync_copy(out_v, out_hbm.at[pl.ds(rows0, RPT), pl.ds(qoff, QW)])


_agg = pl.kernel(
    _sc_body,
    out_type=jax.ShapeDtypeStruct((N, D), jnp.float32),
    mesh=plsc.VectorSubcoreMesh(core_axis_name="c", subcore_axis_name="s"),
    scratch_types=[
        pltpu.VMEM((2 * GPS * 4, QW), jnp.float32),
        pltpu.VMEM((GPS * 16, QW // 2), jnp.int32),
        pltpu.VMEM((RPT, NG), jnp.int32),
        pltpu.VMEM((RPT, QW), jnp.float32),
        pltpu.VMEM((N,), jnp.float32),
        pltpu.VMEM((RPT, QW), jnp.float32),
        pltpu.SemaphoreType.DMA((4,)),
    ],
)


def kernel(node_features, adj_matrix, W, b):
    x = node_features[0]
    a = adj_matrix[0].astype(jnp.int32)
    h, tab, deg, m = pl.pallas_call(
        _prep_body,
        out_shape=[
            jax.ShapeDtypeStruct((N, D), jnp.float32),
            jax.ShapeDtypeStruct((N, D), jnp.float32),
            jax.ShapeDtypeStruct((1, N), jnp.float32),
            jax.ShapeDtypeStruct((N, NG), jnp.int32),
        ],
    )(x, a, W, b.reshape(1, -1))
    out = _agg(tab, m, h, deg.reshape(-1))
    return out[None]
